# baseline (speedup 1.0000x reference)
import functools

import jax
import jax.numpy as jnp
from jax import lax
from jax.experimental import pallas as pl
from jax.experimental.pallas import tpu as pltpu
from jax.experimental.pallas import tpu_sc as plsc

_N, _C, _H, _W = 256, 96, 32, 32
_B, _G = 4, 64
_HW = _H * _W
_NB = 8
_NCELL = _B * 64
_NBAND = _B * 8
_NSTEP = _N + _NBAND


def _csf_body(wl_ref, wh_ref, l_ref, h_ref, out_ref):
    for j in range(_NB):
        acc = lax.dot_general(wl_ref[...], l_ref[j], (((1,), (0,)), ((), ())),
                              preferred_element_type=jnp.float32)
        acc = acc + lax.dot_general(wh_ref[...], h_ref[j], (((1,), (0,)), ((), ())),
                                    preferred_element_type=jnp.float32)
        out_ref[j] = jnp.maximum(acc, 0.0)


def _csf(l3, h3, W_csf):
    wl = W_csf[:_C].T
    wh = W_csf[_C:].T
    return pl.pallas_call(
        _csf_body,
        grid=(_N // _NB,),
        in_specs=[
            pl.BlockSpec((_C, _C), lambda i: (0, 0)),
            pl.BlockSpec((_C, _C), lambda i: (0, 0)),
            pl.BlockSpec((_NB, _C, _HW), lambda i: (i, 0, 0)),
            pl.BlockSpec((_NB, _C, _HW), lambda i: (i, 0, 0)),
        ],
        out_specs=pl.BlockSpec((_NB, _C, _HW), lambda i: (i, 0, 0)),
        out_shape=jax.ShapeDtypeStruct((_N, _C, _HW), jnp.float32),
    )(wl, wh, l3, h3)


def _scatter_body(sb, sgy, sgx, sfetch, sinit, scnt, pred_ref, out_ref):
    s = pl.program_id(0)

    @pl.when(sinit[s] == 1)
    def _zero():
        out_ref[...] = jnp.zeros_like(out_ref)

    @pl.when(sinit[s] == 0)
    def _acc():
        scale = 1.0 / (scnt[s].astype(jnp.float32) + 1e-6)
        val = pred_ref[0] * scale
        gx = sgx[s]
        for g in range(8):
            @pl.when(gx == g)
            def _add(g=g):
                out_ref[0, :, :, g * _W:(g + 1) * _W] = (
                    out_ref[0, :, :, g * _W:(g + 1) * _W] + val)


def _schedule(coords):
    n_idx = jnp.arange(_N, dtype=jnp.int32)
    cell = (n_idx // _G) * 64 + coords[:, 0] * 8 + coords[:, 1]
    counts = jnp.zeros((_NCELL,), jnp.int32).at[cell].add(1)
    order = jnp.argsort(cell).astype(jnp.int32)
    cell_sorted = cell[order]
    band_counts = counts.reshape(_NBAND, 8).sum(axis=1)
    bsum = jnp.cumsum(band_counts) - band_counts
    band_idx = jnp.arange(_NBAND, dtype=jnp.int32)
    run_start = band_idx + bsum
    win_steps = cell_sorted // 8 + 1 + jnp.arange(_N, dtype=jnp.int32)
    step_fetch = jnp.zeros((_NSTEP,), jnp.int32).at[win_steps].set(order)
    first_win = jnp.where(band_counts > 0,
                          order[jnp.minimum(bsum, _N - 1)], 0)
    step_fetch = step_fetch.at[run_start].set(first_win)
    is_init = jnp.zeros((_NSTEP,), jnp.int32).at[run_start].set(1)
    step_band = jnp.repeat(band_idx, band_counts + 1,
                           total_repeat_length=_NSTEP)
    step_b = step_band // 8
    step_gy = step_band % 8
    step_gx = jnp.zeros((_NSTEP,), jnp.int32).at[win_steps].set(cell_sorted % 8)
    step_cnt = jnp.ones((_NSTEP,), jnp.int32).at[win_steps].set(
        counts[cell_sorted])
    return step_b, step_gy, step_gx, step_fetch, is_init, step_cnt


def _stitch(preds4, coords):
    step_b, step_gy, step_gx, step_fetch, is_init, step_cnt = _schedule(coords)
    grid_spec = pltpu.PrefetchScalarGridSpec(
        num_scalar_prefetch=6,
        grid=(_NSTEP,),
        in_specs=[
            pl.BlockSpec((1, _C, _H, _W),
                         lambda s, sb, sgy, sgx, sf, si, sc: (sf[s], 0, 0, 0)),
        ],
        out_specs=pl.BlockSpec((1, _C, _H, _W * 8),
                               lambda s, sb, sgy, sgx, sf, si, sc:
                               (sb[s], 0, sgy[s], 0)),
    )
    return pl.pallas_call(
        _scatter_body,
        grid_spec=grid_spec,
        out_shape=jax.ShapeDtypeStruct((_B, _C, _H * 8, _W * 8), jnp.float32),
    )(step_b, step_gy, step_gx, step_fetch, is_init, step_cnt, preds4)


_CC = 4
_NPASS = _C // _CC
_NLOC = 128
_WPT = 8


def _sc_stitch(preds, cells_local):
    mesh = plsc.VectorSubcoreMesh(core_axis_name="c", subcore_axis_name="s")

    @functools.partial(
        pl.kernel,
        out_type=jax.ShapeDtypeStruct((_B, _C, _H * 8, _W * 8), jnp.float32),
        mesh=mesh,
        compiler_params=pltpu.CompilerParams(use_tc_tiling_on_sc=False),
        scratch_types=[
            pltpu.VMEM_SHARED((_NLOC, _CC, _H, _W), jnp.float32),
            pltpu.VMEM((_WPT, _CC, _H, _W), jnp.float32),
            pltpu.VMEM((_CC, _H, _W), jnp.float32),
            pltpu.VMEM((_CC, _H, _W), jnp.float32),
            pltpu.VMEM((_WPT,), jnp.int32),
            pltpu.VMEM((_WPT, 16), jnp.float32),
            pltpu.VMEM_SHARED((_NLOC, 16), jnp.float32),
            pltpu.VMEM((_WPT, 16), jnp.float32),
        ],
    )
    def stitch(preds_hbm, cells_hbm, out_hbm, acc, win_buf, cell_buf,
               zero_buf, idx_buf, scales_v, cnt_acc, ones_buf):
        core = lax.axis_index("c")
        tid = lax.axis_index("s")
        base_n = core * _NLOC
        my_row0 = tid * _WPT
        pltpu.sync_copy(cells_hbm.at[pl.ds(base_n + my_row0, _WPT)], idx_buf)
        for j in range(_WPT):
            scales_v[j] = jnp.zeros((16,), jnp.float32)
            ones_buf[j] = jnp.full((16,), 1.0, jnp.float32)
        pltpu.sync_copy(scales_v, cnt_acc.at[pl.ds(my_row0, _WPT)])
        plsc.subcore_barrier()
        pltpu.sync_copy(ones_buf, cnt_acc.at[idx_buf], add=True)
        plsc.subcore_barrier()
        pltpu.sync_copy(cnt_acc.at[pl.ds(my_row0, _WPT)], scales_v)
        for j in range(_WPT):
            scales_v[j] = jnp.full((16,), 1.0, jnp.float32) / (
                scales_v[j] + 1e-6)
        for ch in range(_CC):
            def _zb(r, _, ch=ch):
                for u in range(2):
                    zero_buf[ch, r, pl.ds(u * 16, 16)] = jnp.zeros(
                        (16,), jnp.float32)
                return 0
            lax.fori_loop(0, _H, _zb, 0)

        def pass_body(p, _):
            c0 = p * _CC
            for j in range(_WPT):
                pltpu.sync_copy(zero_buf, acc.at[my_row0 + j])
            plsc.subcore_barrier()
            pltpu.sync_copy(
                preds_hbm.at[pl.ds(base_n + my_row0, _WPT), pl.ds(c0, _CC)],
                win_buf)
            pltpu.sync_copy(win_buf, acc.at[idx_buf], add=True)
            plsc.subcore_barrier()
            for j in range(_WPT):
                row = my_row0 + j
                pltpu.sync_copy(acc.at[row], cell_buf)
                scale = scales_v[j]
                for ch in range(_CC):
                    def _sb(r, _, ch=ch):
                        for u in range(2):
                            cell_buf[ch, r, pl.ds(u * 16, 16)] = (
                                cell_buf[ch, r, pl.ds(u * 16, 16)] * scale)
                        return 0
                    lax.fori_loop(0, _H, _sb, 0)
                cell = core * _NLOC + row
                b = cell // 64
                gy = (cell % 64) // 8
                gx = cell % 8
                pltpu.sync_copy(
                    cell_buf,
                    out_hbm.at[b, pl.ds(c0, _CC),
                               pl.ds(gy * _H, _H), pl.ds(gx * _W, _W)])
            return 0

        lax.fori_loop(0, _NPASS, pass_body, 0)

    return stitch(preds, cells_local)


def kernel(l_input_features, h_inputs_features, candidate_windows_mask,
           coords_list, W_csf):
    l3 = l_input_features.reshape(_N, _C, _HW)
    h3 = h_inputs_features.reshape(_N, _C, _HW)
    preds = _csf(l3, h3, W_csf)
    n_idx = jnp.arange(_N, dtype=jnp.int32)
    cells_local = ((n_idx // _G) * 64
                   + coords_list[:, 0] * 8 + coords_list[:, 1]) % _NLOC
    full = _sc_stitch(preds.reshape(_N, _C, _H, _W), cells_local)
    return full, preds.reshape(_N, _C, _H, _W)


# speedup vs baseline: 1.0608x; 1.0608x over previous
import functools

import jax
import jax.numpy as jnp
from jax import lax
from jax.experimental import pallas as pl
from jax.experimental.pallas import tpu as pltpu
from jax.experimental.pallas import tpu_sc as plsc

_N, _C, _H, _W = 256, 96, 32, 32
_B, _G = 4, 64
_HW = _H * _W
_NB = 8
_NCELL = _B * 64
_NBAND = _B * 8
_NSTEP = _N + _NBAND


def _csf_body(wl_ref, wh_ref, sc_ref, l_ref, h_ref, out_ref, outs_ref):
    for j in range(_NB):
        acc = lax.dot_general(wl_ref[...], l_ref[j], (((1,), (0,)), ((), ())),
                              preferred_element_type=jnp.float32)
        acc = acc + lax.dot_general(wh_ref[...], h_ref[j], (((1,), (0,)), ((), ())),
                                    preferred_element_type=jnp.float32)
        acc = jnp.maximum(acc, 0.0)
        out_ref[j] = acc
        outs_ref[j] = acc * sc_ref[j, 0, 0]


def _csf(l3, h3, W_csf, wscale):
    wl = W_csf[:_C].T
    wh = W_csf[_C:].T
    return pl.pallas_call(
        _csf_body,
        grid=(_N // _NB,),
        in_specs=[
            pl.BlockSpec((_C, _C), lambda i: (0, 0)),
            pl.BlockSpec((_C, _C), lambda i: (0, 0)),
            pl.BlockSpec((_NB, 1, 1), lambda i: (i, 0, 0)),
            pl.BlockSpec((_NB, _C, _HW), lambda i: (i, 0, 0)),
            pl.BlockSpec((_NB, _C, _HW), lambda i: (i, 0, 0)),
        ],
        out_specs=[
            pl.BlockSpec((_NB, _C, _HW), lambda i: (i, 0, 0)),
            pl.BlockSpec((_NB, _C, _HW), lambda i: (i, 0, 0)),
        ],
        out_shape=[
            jax.ShapeDtypeStruct((_N, _C, _HW), jnp.float32),
            jax.ShapeDtypeStruct((_N, _C, _HW), jnp.float32),
        ],
    )(wl, wh, wscale, l3, h3)


def _scatter_body(sb, sgy, sgx, sfetch, sinit, scnt, pred_ref, out_ref):
    s = pl.program_id(0)

    @pl.when(sinit[s] == 1)
    def _zero():
        out_ref[...] = jnp.zeros_like(out_ref)

    @pl.when(sinit[s] == 0)
    def _acc():
        scale = 1.0 / (scnt[s].astype(jnp.float32) + 1e-6)
        val = pred_ref[0] * scale
        gx = sgx[s]
        for g in range(8):
            @pl.when(gx == g)
            def _add(g=g):
                out_ref[0, :, :, g * _W:(g + 1) * _W] = (
                    out_ref[0, :, :, g * _W:(g + 1) * _W] + val)


def _schedule(coords):
    n_idx = jnp.arange(_N, dtype=jnp.int32)
    cell = (n_idx // _G) * 64 + coords[:, 0] * 8 + coords[:, 1]
    counts = jnp.zeros((_NCELL,), jnp.int32).at[cell].add(1)
    order = jnp.argsort(cell).astype(jnp.int32)
    cell_sorted = cell[order]
    band_counts = counts.reshape(_NBAND, 8).sum(axis=1)
    bsum = jnp.cumsum(band_counts) - band_counts
    band_idx = jnp.arange(_NBAND, dtype=jnp.int32)
    run_start = band_idx + bsum
    win_steps = cell_sorted // 8 + 1 + jnp.arange(_N, dtype=jnp.int32)
    step_fetch = jnp.zeros((_NSTEP,), jnp.int32).at[win_steps].set(order)
    first_win = jnp.where(band_counts > 0,
                          order[jnp.minimum(bsum, _N - 1)], 0)
    step_fetch = step_fetch.at[run_start].set(first_win)
    is_init = jnp.zeros((_NSTEP,), jnp.int32).at[run_start].set(1)
    step_band = jnp.repeat(band_idx, band_counts + 1,
                           total_repeat_length=_NSTEP)
    step_b = step_band // 8
    step_gy = step_band % 8
    step_gx = jnp.zeros((_NSTEP,), jnp.int32).at[win_steps].set(cell_sorted % 8)
    step_cnt = jnp.ones((_NSTEP,), jnp.int32).at[win_steps].set(
        counts[cell_sorted])
    return step_b, step_gy, step_gx, step_fetch, is_init, step_cnt


def _stitch(preds4, coords):
    step_b, step_gy, step_gx, step_fetch, is_init, step_cnt = _schedule(coords)
    grid_spec = pltpu.PrefetchScalarGridSpec(
        num_scalar_prefetch=6,
        grid=(_NSTEP,),
        in_specs=[
            pl.BlockSpec((1, _C, _H, _W),
                         lambda s, sb, sgy, sgx, sf, si, sc: (sf[s], 0, 0, 0)),
        ],
        out_specs=pl.BlockSpec((1, _C, _H, _W * 8),
                               lambda s, sb, sgy, sgx, sf, si, sc:
                               (sb[s], 0, sgy[s], 0)),
    )
    return pl.pallas_call(
        _scatter_body,
        grid_spec=grid_spec,
        out_shape=jax.ShapeDtypeStruct((_B, _C, _H * 8, _W * 8), jnp.float32),
    )(step_b, step_gy, step_gx, step_fetch, is_init, step_cnt, preds4)


_CC = 4
_NPASS = _C // _CC
_NLOC = 128
_WPT = 8


def _sc_stitch(preds_scaled, cells_local):
    mesh = plsc.VectorSubcoreMesh(core_axis_name="c", subcore_axis_name="s")

    @functools.partial(
        pl.kernel,
        out_type=jax.ShapeDtypeStruct((_B, _C, _H * 8, _W * 8), jnp.float32),
        mesh=mesh,
        compiler_params=pltpu.CompilerParams(use_tc_tiling_on_sc=False),
        scratch_types=[
            pltpu.VMEM_SHARED((_NLOC, _CC, _H, _W), jnp.float32),
            pltpu.VMEM((_WPT, _CC, _H, _W), jnp.float32),
            pltpu.VMEM((_WPT, _CC, _H, _W), jnp.float32),
            pltpu.VMEM((_CC, _H, _W), jnp.float32),
            pltpu.VMEM((_WPT,), jnp.int32),
        ],
    )
    def stitch(preds_hbm, cells_hbm, out_hbm, acc, win_buf, zero_buf,
               cell_buf, idx_buf):
        core = lax.axis_index("c")
        tid = lax.axis_index("s")
        base_n = core * _NLOC
        my_row0 = tid * _WPT
        pltpu.sync_copy(cells_hbm.at[pl.ds(base_n + my_row0, _WPT)], idx_buf)
        for j in range(_WPT):
            for ch in range(_CC):
                def _zb(r, _, j=j, ch=ch):
                    for u in range(2):
                        zero_buf[j, ch, r, pl.ds(u * 16, 16)] = jnp.zeros(
                            (16,), jnp.float32)
                    return 0
                lax.fori_loop(0, _H, _zb, 0)

        def pass_body(p, _):
            c0 = p * _CC
            pltpu.sync_copy(zero_buf, acc.at[pl.ds(my_row0, _WPT)])
            plsc.subcore_barrier()
            pltpu.sync_copy(
                preds_hbm.at[pl.ds(base_n + my_row0, _WPT), pl.ds(c0, _CC)],
                win_buf)
            pltpu.sync_copy(win_buf, acc.at[idx_buf], add=True)
            plsc.subcore_barrier()
            for j in range(_WPT):
                row = my_row0 + j
                cell = core * _NLOC + row
                b = cell // 64
                gy = (cell % 64) // 8
                gx = cell % 8
                pltpu.sync_copy(acc.at[row], cell_buf)
                pltpu.sync_copy(
                    cell_buf,
                    out_hbm.at[b, pl.ds(c0, _CC),
                               pl.ds(gy * _H, _H), pl.ds(gx * _W, _W)])
            return 0

        lax.fori_loop(0, _NPASS, pass_body, 0)

    return stitch(preds_scaled, cells_local)


def kernel(l_input_features, h_inputs_features, candidate_windows_mask,
           coords_list, W_csf):
    l3 = l_input_features.reshape(_N, _C, _HW)
    h3 = h_inputs_features.reshape(_N, _C, _HW)
    n_idx = jnp.arange(_N, dtype=jnp.int32)
    cell = (n_idx // _G) * 64 + coords_list[:, 0] * 8 + coords_list[:, 1]
    counts = jnp.zeros((_NCELL,), jnp.float32).at[cell].add(1.0)
    wscale = (1.0 / (counts + 1e-6))[cell].reshape(_N, 1, 1)
    preds, preds_scaled = _csf(l3, h3, W_csf, wscale)
    full = _sc_stitch(preds_scaled.reshape(_N, _C, _H, _W), cell % _NLOC)
    return full, preds.reshape(_N, _C, _H, _W)


# speedup vs baseline: 1.1379x; 1.0727x over previous
import functools

import jax
import jax.numpy as jnp
from jax import lax
from jax.experimental import pallas as pl
from jax.experimental.pallas import tpu as pltpu
from jax.experimental.pallas import tpu_sc as plsc

_N, _C, _H, _W = 256, 96, 32, 32
_B, _G = 4, 64
_HW = _H * _W
_NB = 8
_NCELL = _B * 64
_NBAND = _B * 8
_NSTEP = _N + _NBAND


def _csf_body(wl_ref, wh_ref, sc_ref, l_ref, h_ref, out_ref, outs_ref):
    for j in range(_NB):
        acc = lax.dot_general(wl_ref[...], l_ref[j], (((1,), (0,)), ((), ())),
                              preferred_element_type=jnp.float32)
        acc = acc + lax.dot_general(wh_ref[...], h_ref[j], (((1,), (0,)), ((), ())),
                                    preferred_element_type=jnp.float32)
        acc = jnp.maximum(acc, 0.0)
        out_ref[j] = acc
        outs_ref[j] = acc * sc_ref[j, 0, 0]


def _csf(l3, h3, W_csf, wscale):
    wl = W_csf[:_C].T
    wh = W_csf[_C:].T
    return pl.pallas_call(
        _csf_body,
        grid=(_N // _NB,),
        in_specs=[
            pl.BlockSpec((_C, _C), lambda i: (0, 0)),
            pl.BlockSpec((_C, _C), lambda i: (0, 0)),
            pl.BlockSpec((_NB, 1, 1), lambda i: (i, 0, 0)),
            pl.BlockSpec((_NB, _C, _HW), lambda i: (i, 0, 0)),
            pl.BlockSpec((_NB, _C, _HW), lambda i: (i, 0, 0)),
        ],
        out_specs=[
            pl.BlockSpec((_NB, _C, _HW), lambda i: (i, 0, 0)),
            pl.BlockSpec((_NB, _C, _HW), lambda i: (i, 0, 0)),
        ],
        out_shape=[
            jax.ShapeDtypeStruct((_N, _C, _HW), jnp.float32),
            jax.ShapeDtypeStruct((_N, _C, _HW), jnp.float32),
        ],
    )(wl, wh, wscale, l3, h3)


def _scatter_body(sb, sgy, sgx, sfetch, sinit, scnt, pred_ref, out_ref):
    s = pl.program_id(0)

    @pl.when(sinit[s] == 1)
    def _zero():
        out_ref[...] = jnp.zeros_like(out_ref)

    @pl.when(sinit[s] == 0)
    def _acc():
        scale = 1.0 / (scnt[s].astype(jnp.float32) + 1e-6)
        val = pred_ref[0] * scale
        gx = sgx[s]
        for g in range(8):
            @pl.when(gx == g)
            def _add(g=g):
                out_ref[0, :, :, g * _W:(g + 1) * _W] = (
                    out_ref[0, :, :, g * _W:(g + 1) * _W] + val)


def _schedule(coords):
    n_idx = jnp.arange(_N, dtype=jnp.int32)
    cell = (n_idx // _G) * 64 + coords[:, 0] * 8 + coords[:, 1]
    counts = jnp.zeros((_NCELL,), jnp.int32).at[cell].add(1)
    order = jnp.argsort(cell).astype(jnp.int32)
    cell_sorted = cell[order]
    band_counts = counts.reshape(_NBAND, 8).sum(axis=1)
    bsum = jnp.cumsum(band_counts) - band_counts
    band_idx = jnp.arange(_NBAND, dtype=jnp.int32)
    run_start = band_idx + bsum
    win_steps = cell_sorted // 8 + 1 + jnp.arange(_N, dtype=jnp.int32)
    step_fetch = jnp.zeros((_NSTEP,), jnp.int32).at[win_steps].set(order)
    first_win = jnp.where(band_counts > 0,
                          order[jnp.minimum(bsum, _N - 1)], 0)
    step_fetch = step_fetch.at[run_start].set(first_win)
    is_init = jnp.zeros((_NSTEP,), jnp.int32).at[run_start].set(1)
    step_band = jnp.repeat(band_idx, band_counts + 1,
                           total_repeat_length=_NSTEP)
    step_b = step_band // 8
    step_gy = step_band % 8
    step_gx = jnp.zeros((_NSTEP,), jnp.int32).at[win_steps].set(cell_sorted % 8)
    step_cnt = jnp.ones((_NSTEP,), jnp.int32).at[win_steps].set(
        counts[cell_sorted])
    return step_b, step_gy, step_gx, step_fetch, is_init, step_cnt


def _stitch(preds4, coords):
    step_b, step_gy, step_gx, step_fetch, is_init, step_cnt = _schedule(coords)
    grid_spec = pltpu.PrefetchScalarGridSpec(
        num_scalar_prefetch=6,
        grid=(_NSTEP,),
        in_specs=[
            pl.BlockSpec((1, _C, _H, _W),
                         lambda s, sb, sgy, sgx, sf, si, sc: (sf[s], 0, 0, 0)),
        ],
        out_specs=pl.BlockSpec((1, _C, _H, _W * 8),
                               lambda s, sb, sgy, sgx, sf, si, sc:
                               (sb[s], 0, sgy[s], 0)),
    )
    return pl.pallas_call(
        _scatter_body,
        grid_spec=grid_spec,
        out_shape=jax.ShapeDtypeStruct((_B, _C, _H * 8, _W * 8), jnp.float32),
    )(step_b, step_gy, step_gx, step_fetch, is_init, step_cnt, preds4)


_CC = 4
_NPASS = _C // _CC
_NLOC = 128
_WPT = 8


def _sc_stitch(preds_scaled, cells_local):
    mesh = plsc.VectorSubcoreMesh(core_axis_name="c", subcore_axis_name="s")

    @functools.partial(
        pl.kernel,
        out_type=jax.ShapeDtypeStruct((_B, _C, 32, 2, 8, 128), jnp.float32),
        mesh=mesh,
        compiler_params=pltpu.CompilerParams(use_tc_tiling_on_sc=False),
        scratch_types=[
            pltpu.VMEM_SHARED((_NLOC, _CC, _H, _W), jnp.float32),
            pltpu.VMEM((_WPT, _CC, _H, _W), jnp.float32),
            pltpu.VMEM((_WPT, _CC, _H, _W), jnp.float32),
            pltpu.VMEM((_CC, _H, _W), jnp.float32),
            pltpu.VMEM((_WPT,), jnp.int32),
        ],
    )
    def stitch(preds_hbm, cells_hbm, out_hbm, acc, win_buf, zero_buf,
               cell_buf, idx_buf):
        core = lax.axis_index("c")
        tid = lax.axis_index("s")
        base_n = core * _NLOC
        my_row0 = tid * _WPT
        pltpu.sync_copy(cells_hbm.at[pl.ds(base_n + my_row0, _WPT)], idx_buf)
        for j in range(_WPT):
            for ch in range(_CC):
                def _zb(r, _, j=j, ch=ch):
                    for u in range(2):
                        zero_buf[j, ch, r, pl.ds(u * 16, 16)] = jnp.zeros(
                            (16,), jnp.float32)
                    return 0
                lax.fori_loop(0, _H, _zb, 0)

        def pass_body(p, _):
            c0 = p * _CC
            pltpu.sync_copy(zero_buf, acc.at[pl.ds(my_row0, _WPT)])
            plsc.subcore_barrier()
            pltpu.sync_copy(
                preds_hbm.at[pl.ds(base_n + my_row0, _WPT), pl.ds(c0, _CC)],
                win_buf)
            pltpu.sync_copy(win_buf, acc.at[idx_buf], add=True)
            plsc.subcore_barrier()
            for j in range(_WPT):
                row = my_row0 + j
                cell = core * _NLOC + row
                b = cell // 64
                gy = (cell % 64) // 8
                gx = cell % 8
                tc = gx // 4
                lane0 = (gx % 4) * 32
                pltpu.sync_copy(acc.at[row], cell_buf)
                for tr in range(4):
                    pltpu.sync_copy(
                        cell_buf.at[:, pl.ds(tr * 8, 8), :],
                        out_hbm.at[b, pl.ds(c0, _CC), gy * 4 + tr, tc,
                                   slice(None), pl.ds(lane0, 32)])
            return 0

        lax.fori_loop(0, _NPASS, pass_body, 0)

    return stitch(preds_scaled, cells_local)


def kernel(l_input_features, h_inputs_features, candidate_windows_mask,
           coords_list, W_csf):
    l3 = l_input_features.reshape(_N, _C, _HW)
    h3 = h_inputs_features.reshape(_N, _C, _HW)
    n_idx = jnp.arange(_N, dtype=jnp.int32)
    cell = (n_idx // _G) * 64 + coords_list[:, 0] * 8 + coords_list[:, 1]
    counts = jnp.zeros((_NCELL,), jnp.float32).at[cell].add(1.0)
    wscale = (1.0 / (counts + 1e-6))[cell].reshape(_N, 1, 1)
    preds, preds_scaled = _csf(l3, h3, W_csf, wscale)
    full6 = _sc_stitch(preds_scaled.reshape(_N, _C, _H, _W), cell % _NLOC)
    full = full6.transpose(0, 1, 2, 4, 3, 5).reshape(_B, _C, _H * 8, _W * 8)
    return full, preds.reshape(_N, _C, _H, _W)
